# baseline (speedup 1.0000x reference)
import functools

import jax
import jax.numpy as jnp
from jax import lax
from jax.experimental import pallas as pl
from jax.experimental.pallas import tpu as pltpu
from jax.experimental.pallas import tpu_sc as plsc

B = 8
M = 2048
D = 1024
NG = 1024
SEQ = 2048

NW = 32
WPB = NW // B
RPW = NG // WPB
CR = 16
NCH = RPW // CR
NJ = D // 16
_NEG = -3.0e38


def _qprobe_body(h_ref, q_ref):
    q_ref[0, 0, :] = jnp.mean(h_ref[0], axis=0)


def _qprobe(h):
    return pl.pallas_call(
        _qprobe_body,
        grid=(B,),
        in_specs=[pl.BlockSpec((1, SEQ, D), lambda b: (b, 0, 0))],
        out_specs=pl.BlockSpec((1, 1, D), lambda b: (b, 0, 0)),
        out_shape=jax.ShapeDtypeStruct((B, 1, D), jnp.float32),
    )(h)


def _sc_attn_body(kf_ref, vf_ref, q_ref, idx_ref, outp_ref, outs_ref,
                  idx_v, q_v, kb0, kb1, vb0, vb1, pk, pv, sbuf,
                  sem_k0, sem_k1, sem_v0, sem_v1):
    wid = lax.axis_index("c") * 16 + lax.axis_index("s")
    b = wid // WPB
    w = wid % WPB
    scale = D ** (-0.5)
    iota = lax.broadcasted_iota(jnp.int32, (16,), 0)

    def perm(v, pidx):
        return lax.gather(
            v, pidx[:, None],
            dimension_numbers=lax.GatherDimensionNumbers(
                offset_dims=(), collapsed_slice_dims=(0,),
                start_index_map=(0,)),
            slice_sizes=(1,),
            mode=lax.GatherScatterMode.PROMISE_IN_BOUNDS)

    def bsum(v):
        for k in (8, 4, 2, 1):
            v = v + perm(v, iota ^ k)
        return v

    def bmax(v):
        for k in (8, 4, 2, 1):
            v = jnp.maximum(v, perm(v, iota ^ k))
        return v

    pltpu.sync_copy(idx_ref.at[b, pl.ds(w * NCH, NCH)], idx_v)
    pltpu.sync_copy(q_ref.at[b], q_v)

    zero = jnp.zeros((16,), jnp.float32)
    for j in range(NJ):
        pk[pl.ds(j * 16, 16)] = zero
        pv[pl.ds(j * 16, 16)] = zero

    def issue(c, table_ref, buf, sem):
        pltpu.async_copy(table_ref.at[idx_v.at[c]], buf, sem)

    def drain(table_ref, buf, sem):
        pltpu.make_async_copy(table_ref.at[pl.ds(0, CR)], buf, sem).wait()

    def chunk(buf, p_ref, m_vec, s_vec):
        l_vec = jnp.full((16,), _NEG, jnp.float32)
        for r in range(CR):
            def jdot(j, acc):
                return acc + q_v[pl.ds(j * 16, 16)] * buf[r, pl.ds(j * 16, 16)]
            acc = lax.fori_loop(1, NJ, jdot,
                                q_v[pl.ds(0, 16)] * buf[r, pl.ds(0, 16)])
            l_vec = jnp.where(iota == r, bsum(acc) * scale, l_vec)
        new_m_vec = jnp.maximum(m_vec, bmax(l_vec))
        w_vec = jnp.exp(l_vec - new_m_vec)
        c_vec = jnp.exp(m_vec - new_m_vec)
        s_new = s_vec * c_vec + bsum(w_vec)
        c_s = c_vec[0]
        w_sc = [w_vec[r] for r in range(CR)]

        def pacc(j, _):
            pvv = p_ref[pl.ds(j * 16, 16)] * c_s
            for r in range(CR):
                pvv = pvv + w_sc[r] * buf[r, pl.ds(j * 16, 16)]
            p_ref[pl.ds(j * 16, 16)] = pvv
            return 0
        lax.fori_loop(0, NJ, pacc, 0)
        return new_m_vec, s_new

    issue(0, kf_ref, kb0, sem_k0)
    issue(0, vf_ref, vb0, sem_v0)
    issue(1, kf_ref, kb1, sem_k1)
    issue(1, vf_ref, vb1, sem_v1)

    def body(i, carry):
        mk, sk, mv, sv = carry
        c0 = 2 * i
        c1 = 2 * i + 1
        drain(kf_ref, kb0, sem_k0)
        mk, sk = chunk(kb0, pk, mk, sk)
        drain(vf_ref, vb0, sem_v0)
        mv, sv = chunk(vb0, pv, mv, sv)

        @pl.when(c0 + 2 < NCH)
        def _():
            issue(c0 + 2, kf_ref, kb0, sem_k0)
            issue(c0 + 2, vf_ref, vb0, sem_v0)

        drain(kf_ref, kb1, sem_k1)
        mk, sk = chunk(kb1, pk, mk, sk)
        drain(vf_ref, vb1, sem_v1)
        mv, sv = chunk(vb1, pv, mv, sv)

        @pl.when(c1 + 2 < NCH)
        def _():
            issue(c1 + 2, kf_ref, kb1, sem_k1)
            issue(c1 + 2, vf_ref, vb1, sem_v1)
        return mk, sk, mv, sv

    neg = jnp.full((16,), _NEG, jnp.float32)
    mk, sk, mv, sv = lax.fori_loop(
        0, NCH // 2, body, (neg, zero, neg, zero))

    st = jnp.where(iota == 0, mk, jnp.full((16,), 0.0, jnp.float32))
    st = jnp.where(iota == 1, sk, st)
    st = jnp.where(iota == 2, mv, st)
    st = jnp.where(iota == 3, sv, st)
    sbuf[...] = st
    pltpu.sync_copy(pk, outp_ref.at[wid, 0])
    pltpu.sync_copy(pv, outp_ref.at[wid, 1])
    pltpu.sync_copy(sbuf, outs_ref.at[wid])


def _sc_attn(Kf, Vf, q2, idx4):
    mesh = plsc.VectorSubcoreMesh(core_axis_name="c", subcore_axis_name="s")
    f = pl.kernel(
        _sc_attn_body,
        mesh=mesh,
        out_type=[
            jax.ShapeDtypeStruct((NW, 2, D), jnp.float32),
            jax.ShapeDtypeStruct((NW, 16), jnp.float32),
        ],
        scratch_types=[
            pltpu.VMEM((NCH, CR), jnp.int32),
            pltpu.VMEM((D,), jnp.float32),
            pltpu.VMEM((CR, D), jnp.float32),
            pltpu.VMEM((CR, D), jnp.float32),
            pltpu.VMEM((CR, D), jnp.float32),
            pltpu.VMEM((CR, D), jnp.float32),
            pltpu.VMEM((D,), jnp.float32),
            pltpu.VMEM((D,), jnp.float32),
            pltpu.VMEM((16,), jnp.float32),
            pltpu.SemaphoreType.DMA,
            pltpu.SemaphoreType.DMA,
            pltpu.SemaphoreType.DMA,
            pltpu.SemaphoreType.DMA,
        ],
    )
    return f(Kf, Vf, q2, idx4)


def _gates_body(stats_ref, p_ref, q_ref, mom_ref, idx_ref,
                wk0_ref, wk1_ref, bk_ref, wv0_ref, wv1_ref, bv_ref,
                leta_ref, lalpha_ref, gk_ref, gv_ref, nm_ref):
    q = q_ref[0, 0, :]
    stats = stats_ref[0]

    def pred_one(mcol, scol, prow):
        m_i = stats[:, mcol]
        s_i = stats[:, scol]
        mg = jnp.max(m_i)
        c_i = jnp.exp(m_i - mg)
        den = jnp.sum(c_i * s_i)
        p_i = p_ref[0, :, prow, :]
        return jnp.dot(c_i, p_i, preferred_element_type=jnp.float32) / den

    kp = pred_one(0, 1, 0)
    vp = pred_one(2, 3, 1)
    ks = jnp.mean((kp - q) ** 2)
    vs = jnp.mean((vp - q) ** 2)
    alpha = jax.nn.sigmoid(lalpha_ref[0, 0, 0])
    comb = alpha * ks + (1.0 - alpha) * vs
    eta = jax.nn.sigmoid(leta_ref[0, 0, 0])
    nm = eta * mom_ref[0, 0, 0] + (1.0 - eta) * comb
    nm_ref[0] = jnp.full((1, 1), nm, jnp.float32)

    idx = idx_ref[0, 0, :]
    iom = lax.broadcasted_iota(jnp.int32, (NG, M), 1)
    ion = lax.broadcasted_iota(jnp.int32, (NG, M), 0) + 1
    ln1 = jnp.max(jnp.where(idx[:, None] == iom, ion, 0), axis=0)
    ion1 = lax.broadcasted_iota(jnp.int32, (M, NG), 1) + 1
    Bm = (ln1[:, None] == ion1).astype(jnp.float32)
    inactive = (ln1 == 0).astype(jnp.float32)

    def one(w0_ref, w1_ref, b_ref, g_ref):
        gate_n = jax.nn.sigmoid(ks * w0_ref[0, 0, :] + nm * w1_ref[0, 0, :]
                                + b_ref[0, 0, :])
        g_ref[0, 0, :] = jnp.dot(Bm, gate_n,
                                 preferred_element_type=jnp.float32) + inactive

    one(wk0_ref, wk1_ref, bk_ref, gk_ref)
    one(wv0_ref, wv1_ref, bv_ref, gv_ref)


def _gates(stats, p, q, mom, idx, wk0, wk1, bk, wv0, wv1, bv, leta, lalpha):
    bcast = pl.BlockSpec((1, 1, NG), lambda b: (0, 0, 0))
    scal = pl.BlockSpec((1, 1, 1), lambda b: (0, 0, 0))
    return pl.pallas_call(
        _gates_body,
        grid=(B,),
        in_specs=[
            pl.BlockSpec((1, WPB, 16), lambda b: (b, 0, 0)),
            pl.BlockSpec((1, WPB, 2, D), lambda b: (b, 0, 0, 0)),
            pl.BlockSpec((1, 1, D), lambda b: (b, 0, 0)),
            pl.BlockSpec((1, 1, 1), lambda b: (b, 0, 0)),
            pl.BlockSpec((1, 1, NG), lambda b: (b, 0, 0)),
            bcast, bcast, bcast, bcast, bcast, bcast,
            scal, scal,
        ],
        out_specs=[
            pl.BlockSpec((1, 1, M), lambda b: (b, 0, 0)),
            pl.BlockSpec((1, 1, M), lambda b: (b, 0, 0)),
            pl.BlockSpec((1, 1, 1), lambda b: (b, 0, 0)),
        ],
        out_shape=[
            jax.ShapeDtypeStruct((B, 1, M), jnp.float32),
            jax.ShapeDtypeStruct((B, 1, M), jnp.float32),
            jax.ShapeDtypeStruct((B, 1, 1), jnp.float32),
        ],
    )(stats, p, q, mom, idx, wk0, wk1, bk, wv0, wv1, bv, leta, lalpha)


BM_BLEND = 512


def _blend_body(kc_ref, kp_ref, vc_ref, vp_ref, gk_ref, gv_ref,
                ko_ref, vo_ref):
    j = pl.program_id(1)
    gk = gk_ref[0, 0, pl.ds(j * BM_BLEND, BM_BLEND)][:, None]
    gv = gv_ref[0, 0, pl.ds(j * BM_BLEND, BM_BLEND)][:, None]
    ko_ref[0] = kc_ref[0] * gk + kp_ref[0] * (1.0 - gk)
    vo_ref[0] = vc_ref[0] * gv + vp_ref[0] * (1.0 - gv)


def _blend(K_curr, K_prev, V_curr, V_prev, gk, gv):
    big = pl.BlockSpec((1, BM_BLEND, D), lambda b, j: (b, j, 0))
    row = pl.BlockSpec((1, 1, M), lambda b, j: (b, 0, 0))
    return pl.pallas_call(
        _blend_body,
        grid=(B, M // BM_BLEND),
        in_specs=[big, big, big, big, row, row],
        out_specs=[big, big],
        out_shape=[
            jax.ShapeDtypeStruct((B, M, D), jnp.float32),
            jax.ShapeDtypeStruct((B, M, D), jnp.float32),
        ],
    )(K_curr, K_prev, V_curr, V_prev, gk, gv)


def kernel(K_curr, V_curr, K_prev, V_prev, h, momentum, active_idx,
           Wk, bk, Wv, bv, logit_eta, surprise_logit_alpha):
    idx32 = active_idx.astype(jnp.int32)
    idx = idx32.reshape(B, 1, NG)
    q = _qprobe(h)
    Kf = K_curr.reshape(B * M, D)
    Vf = V_curr.reshape(B * M, D)
    idx4 = (idx32 + (jnp.arange(B, dtype=jnp.int32) * M)[:, None]
            ).reshape(B, NG // CR, CR)
    outp, outs = _sc_attn(Kf, Vf, q.reshape(B, D), idx4)
    stats = outs.reshape(B, WPB, 16)
    p = outp.reshape(B, WPB, 2, D)
    wk0 = Wk[:, 0].reshape(1, 1, NG)
    wk1 = Wk[:, 1].reshape(1, 1, NG)
    wv0 = Wv[:, 0].reshape(1, 1, NG)
    wv1 = Wv[:, 1].reshape(1, 1, NG)
    gk, gv, nm = _gates(stats, p, q, momentum.reshape(B, 1, 1), idx,
                        wk0, wk1, bk.reshape(1, 1, NG),
                        wv0, wv1, bv.reshape(1, 1, NG),
                        jnp.reshape(logit_eta, (1, 1, 1)),
                        jnp.reshape(surprise_logit_alpha, (1, 1, 1)))
    K_out, V_out = _blend(K_curr, K_prev, V_curr, V_prev, gk, gv)
    return (K_out, V_out, nm.reshape(B, 1))


# speedup vs baseline: 1.3946x; 1.3946x over previous
import functools

import jax
import jax.numpy as jnp
from jax import lax
from jax.experimental import pallas as pl
from jax.experimental.pallas import tpu as pltpu
from jax.experimental.pallas import tpu_sc as plsc

B = 8
M = 2048
D = 1024
NG = 1024
SEQ = 2048

NW = 32
WPB = NW // B
RPW = NG // WPB
CR = 16
NCH = RPW // CR
NJ = D // 16
_NEG = -3.0e38


def _qprobe_body(h_ref, q_ref):
    q_ref[0, 0, :] = jnp.mean(h_ref[0], axis=0)


def _qprobe(h):
    return pl.pallas_call(
        _qprobe_body,
        grid=(B,),
        in_specs=[pl.BlockSpec((1, SEQ, D), lambda b: (b, 0, 0))],
        out_specs=pl.BlockSpec((1, 1, D), lambda b: (b, 0, 0)),
        out_shape=jax.ShapeDtypeStruct((B, 1, D), jnp.float32),
    )(h)


NBUF = 4
LAG = 2
NCHT = 2 * NCH


def _sc_gather_body(kf_ref, vf_ref, idx_ref, ko_ref, vo_ref,
                    idx_v, b0, b1, b2, b3,
                    g0, g1, g2, g3, p0, p1, p2, p3):
    wid = lax.axis_index("c") * 16 + lax.axis_index("s")
    b = wid // WPB
    w = wid % WPB
    bufs = [b0, b1, b2, b3]
    gsem = [g0, g1, g2, g3]
    psem = [p0, p1, p2, p3]

    pltpu.sync_copy(idx_ref.at[b, pl.ds(w * NCH, NCH)], idx_v)

    def src_tab(t):
        return kf_ref if t < NCH else vf_ref

    def out_slice(t):
        c = t % NCH
        ref = ko_ref if t < NCH else vo_ref
        return ref.at[wid, pl.ds(c * CR, CR)]

    for t in range(NCHT + LAG):
        if t < NCHT:
            s = t % NBUF
            if t >= NBUF:
                pltpu.make_async_copy(bufs[s], out_slice(t - NBUF),
                                      psem[s]).wait()
            pltpu.async_copy(src_tab(t).at[idx_v.at[t % NCH]], bufs[s],
                             gsem[s])
        u = t - LAG
        if 0 <= u < NCHT:
            s = u % NBUF
            pltpu.make_async_copy(src_tab(u).at[pl.ds(0, CR)], bufs[s],
                                  gsem[s]).wait()
            pltpu.async_copy(bufs[s], out_slice(u), psem[s])

    for u in range(NCHT - NBUF, NCHT):
        s = u % NBUF
        pltpu.make_async_copy(bufs[s], out_slice(u), psem[s]).wait()


def _sc_gather(Kf, Vf, idx4):
    mesh = plsc.VectorSubcoreMesh(core_axis_name="c", subcore_axis_name="s")
    f = pl.kernel(
        _sc_gather_body,
        mesh=mesh,
        out_type=[
            jax.ShapeDtypeStruct((NW, RPW, D), jnp.float32),
            jax.ShapeDtypeStruct((NW, RPW, D), jnp.float32),
        ],
        scratch_types=(
            [pltpu.VMEM((NCH, CR), jnp.int32)]
            + [pltpu.VMEM((CR, D), jnp.float32)] * NBUF
            + [pltpu.SemaphoreType.DMA] * (2 * NBUF)
        ),
    )
    return f(Kf, Vf, idx4)


def _tcattn_body(ka_ref, va_ref, q_ref, kp_ref, vp_ref):
    scale = D ** (-0.5)
    q = q_ref[0, 0, :]

    def one(a_ref, o_ref):
        a = a_ref[0]
        logit = jnp.dot(a, q, preferred_element_type=jnp.float32) * scale
        e = jnp.exp(logit - jnp.max(logit))
        attn = e / jnp.sum(e)
        o_ref[0, 0, :] = jnp.dot(attn, a, preferred_element_type=jnp.float32)

    one(ka_ref, kp_ref)
    one(va_ref, vp_ref)


def _tcattn(Ka, Va, q):
    return pl.pallas_call(
        _tcattn_body,
        grid=(B,),
        in_specs=[
            pl.BlockSpec((1, NG, D), lambda b: (b, 0, 0)),
            pl.BlockSpec((1, NG, D), lambda b: (b, 0, 0)),
            pl.BlockSpec((1, 1, D), lambda b: (b, 0, 0)),
        ],
        out_specs=[
            pl.BlockSpec((1, 1, D), lambda b: (b, 0, 0)),
            pl.BlockSpec((1, 1, D), lambda b: (b, 0, 0)),
        ],
        out_shape=[
            jax.ShapeDtypeStruct((B, 1, D), jnp.float32),
            jax.ShapeDtypeStruct((B, 1, D), jnp.float32),
        ],
    )(Ka, Va, q)


def _gates_body(kp_ref, vp_ref, q_ref, mom_ref, idx_ref,
                wk0_ref, wk1_ref, bk_ref, wv0_ref, wv1_ref, bv_ref,
                leta_ref, lalpha_ref, gk_ref, gv_ref, nm_ref):
    q = q_ref[0, 0, :]
    kp = kp_ref[0, 0, :]
    vp = vp_ref[0, 0, :]
    ks = jnp.mean((kp - q) ** 2)
    vs = jnp.mean((vp - q) ** 2)
    alpha = jax.nn.sigmoid(lalpha_ref[0, 0, 0])
    comb = alpha * ks + (1.0 - alpha) * vs
    eta = jax.nn.sigmoid(leta_ref[0, 0, 0])
    nm = eta * mom_ref[0, 0, 0] + (1.0 - eta) * comb
    nm_ref[0] = jnp.full((1, 1), nm, jnp.float32)

    idx = idx_ref[0, 0, :]
    iom = lax.broadcasted_iota(jnp.int32, (NG, M), 1)
    ion = lax.broadcasted_iota(jnp.int32, (NG, M), 0) + 1
    ln1 = jnp.max(jnp.where(idx[:, None] == iom, ion, 0), axis=0)
    ion1 = lax.broadcasted_iota(jnp.int32, (M, NG), 1) + 1
    Bm = (ln1[:, None] == ion1).astype(jnp.float32)
    inactive = (ln1 == 0).astype(jnp.float32)

    def one(w0_ref, w1_ref, b_ref, g_ref):
        gate_n = jax.nn.sigmoid(ks * w0_ref[0, 0, :] + nm * w1_ref[0, 0, :]
                                + b_ref[0, 0, :])
        g_ref[0, 0, :] = jnp.dot(Bm, gate_n,
                                 preferred_element_type=jnp.float32) + inactive

    one(wk0_ref, wk1_ref, bk_ref, gk_ref)
    one(wv0_ref, wv1_ref, bv_ref, gv_ref)


def _gates(kp, vp, q, mom, idx, wk0, wk1, bk, wv0, wv1, bv, leta, lalpha):
    bcast = pl.BlockSpec((1, 1, NG), lambda b: (0, 0, 0))
    scal = pl.BlockSpec((1, 1, 1), lambda b: (0, 0, 0))
    return pl.pallas_call(
        _gates_body,
        grid=(B,),
        in_specs=[
            pl.BlockSpec((1, 1, D), lambda b: (b, 0, 0)),
            pl.BlockSpec((1, 1, D), lambda b: (b, 0, 0)),
            pl.BlockSpec((1, 1, D), lambda b: (b, 0, 0)),
            pl.BlockSpec((1, 1, 1), lambda b: (b, 0, 0)),
            pl.BlockSpec((1, 1, NG), lambda b: (b, 0, 0)),
            bcast, bcast, bcast, bcast, bcast, bcast,
            scal, scal,
        ],
        out_specs=[
            pl.BlockSpec((1, 1, M), lambda b: (b, 0, 0)),
            pl.BlockSpec((1, 1, M), lambda b: (b, 0, 0)),
            pl.BlockSpec((1, 1, 1), lambda b: (b, 0, 0)),
        ],
        out_shape=[
            jax.ShapeDtypeStruct((B, 1, M), jnp.float32),
            jax.ShapeDtypeStruct((B, 1, M), jnp.float32),
            jax.ShapeDtypeStruct((B, 1, 1), jnp.float32),
        ],
    )(kp, vp, q, mom, idx, wk0, wk1, bk, wv0, wv1, bv, leta, lalpha)


BM_BLEND = 512


def _blend_body(kc_ref, kp_ref, vc_ref, vp_ref, gk_ref, gv_ref,
                ko_ref, vo_ref):
    j = pl.program_id(1)
    gk = gk_ref[0, 0, pl.ds(j * BM_BLEND, BM_BLEND)][:, None]
    gv = gv_ref[0, 0, pl.ds(j * BM_BLEND, BM_BLEND)][:, None]
    ko_ref[0] = kc_ref[0] * gk + kp_ref[0] * (1.0 - gk)
    vo_ref[0] = vc_ref[0] * gv + vp_ref[0] * (1.0 - gv)


def _blend(K_curr, K_prev, V_curr, V_prev, gk, gv):
    big = pl.BlockSpec((1, BM_BLEND, D), lambda b, j: (b, j, 0))
    row = pl.BlockSpec((1, 1, M), lambda b, j: (b, 0, 0))
    return pl.pallas_call(
        _blend_body,
        grid=(B, M // BM_BLEND),
        in_specs=[big, big, big, big, row, row],
        out_specs=[big, big],
        out_shape=[
            jax.ShapeDtypeStruct((B, M, D), jnp.float32),
            jax.ShapeDtypeStruct((B, M, D), jnp.float32),
        ],
    )(K_curr, K_prev, V_curr, V_prev, gk, gv)


def kernel(K_curr, V_curr, K_prev, V_prev, h, momentum, active_idx,
           Wk, bk, Wv, bv, logit_eta, surprise_logit_alpha):
    idx32 = active_idx.astype(jnp.int32)
    idx = idx32.reshape(B, 1, NG)
    Kf = K_curr.reshape(B * M, D)
    Vf = V_curr.reshape(B * M, D)
    idx4 = (idx32 + (jnp.arange(B, dtype=jnp.int32) * M)[:, None]
            ).reshape(B, NG // CR, CR)
    Ko, Vo = _sc_gather(Kf, Vf, idx4)
    q = _qprobe(h)
    kp, vp = _tcattn(Ko.reshape(B, NG, D), Vo.reshape(B, NG, D), q)
    wk0 = Wk[:, 0].reshape(1, 1, NG)
    wk1 = Wk[:, 1].reshape(1, 1, NG)
    wv0 = Wv[:, 0].reshape(1, 1, NG)
    wv1 = Wv[:, 1].reshape(1, 1, NG)
    gk, gv, nm = _gates(kp, vp, q, momentum.reshape(B, 1, 1), idx,
                        wk0, wk1, bk.reshape(1, 1, NG),
                        wv0, wv1, bv.reshape(1, 1, NG),
                        jnp.reshape(logit_eta, (1, 1, 1)),
                        jnp.reshape(surprise_logit_alpha, (1, 1, 1)))
    K_out, V_out = _blend(K_curr, K_prev, V_curr, V_prev, gk, gv)
    return (K_out, V_out, nm.reshape(B, 1))


# speedup vs baseline: 1.4499x; 1.0397x over previous
import functools

import jax
import jax.numpy as jnp
from jax import lax
from jax.experimental import pallas as pl
from jax.experimental.pallas import tpu as pltpu
from jax.experimental.pallas import tpu_sc as plsc

B = 8
M = 2048
D = 1024
NG = 1024
SEQ = 2048

NW = 32
WPB = NW // B
RPW = NG // WPB
CR = 16
NCH = RPW // CR
NJ = D // 16
_NEG = -3.0e38


def _qprobe_body(h_ref, q_ref):
    q_ref[0, 0, :] = jnp.mean(h_ref[0], axis=0)


def _qprobe(h):
    return pl.pallas_call(
        _qprobe_body,
        grid=(B,),
        in_specs=[pl.BlockSpec((1, SEQ, D), lambda b: (b, 0, 0))],
        out_specs=pl.BlockSpec((1, 1, D), lambda b: (b, 0, 0)),
        out_shape=jax.ShapeDtypeStruct((B, 1, D), jnp.float32),
    )(h)


NBUF = 4
LAG = 2
NCHT = 2 * NCH


def _sc_gather_body(kf_ref, vf_ref, idx_ref, ko_ref, vo_ref,
                    idx_v, b0, b1, b2, b3,
                    g0, g1, g2, g3, p0, p1, p2, p3):
    wid = lax.axis_index("c") * 16 + lax.axis_index("s")
    b = wid // WPB
    w = wid % WPB
    bufs = [b0, b1, b2, b3]
    gsem = [g0, g1, g2, g3]
    psem = [p0, p1, p2, p3]

    pltpu.sync_copy(idx_ref.at[b, pl.ds(w * NCH, NCH)], idx_v)

    def src_tab(t):
        return kf_ref if t < NCH else vf_ref

    def out_slice(t):
        c = t % NCH
        ref = ko_ref if t < NCH else vo_ref
        return ref.at[wid, pl.ds(c * CR, CR)]

    for t in range(NCHT + LAG):
        if t < NCHT:
            s = t % NBUF
            if t >= NBUF:
                pltpu.make_async_copy(bufs[s], out_slice(t - NBUF),
                                      psem[s]).wait()
            pltpu.async_copy(src_tab(t).at[idx_v.at[t % NCH]], bufs[s],
                             gsem[s])
        u = t - LAG
        if 0 <= u < NCHT:
            s = u % NBUF
            pltpu.make_async_copy(src_tab(u).at[pl.ds(0, CR)], bufs[s],
                                  gsem[s]).wait()
            pltpu.async_copy(bufs[s], out_slice(u), psem[s])

    for u in range(NCHT - NBUF, NCHT):
        s = u % NBUF
        pltpu.make_async_copy(bufs[s], out_slice(u), psem[s]).wait()


def _sc_gather(Kf, Vf, idx4):
    mesh = plsc.VectorSubcoreMesh(core_axis_name="c", subcore_axis_name="s")
    f = pl.kernel(
        _sc_gather_body,
        mesh=mesh,
        out_type=[
            jax.ShapeDtypeStruct((NW, RPW, D), jnp.float32),
            jax.ShapeDtypeStruct((NW, RPW, D), jnp.float32),
        ],
        scratch_types=(
            [pltpu.VMEM((NCH, CR), jnp.int32)]
            + [pltpu.VMEM((CR, D), jnp.float32)] * NBUF
            + [pltpu.SemaphoreType.DMA] * (2 * NBUF)
        ),
    )
    return f(Kf, Vf, idx4)


def _lastn_body(idx_ref, ln_ref):
    idx = idx_ref[0, 0, :]
    iom = lax.broadcasted_iota(jnp.int32, (NG, M), 1)
    ion = lax.broadcasted_iota(jnp.int32, (NG, M), 0) + 1
    ln_ref[0, 0, :] = jnp.max(jnp.where(idx[:, None] == iom, ion, 0), axis=0)


def _lastn(idx):
    return pl.pallas_call(
        _lastn_body,
        grid=(B,),
        in_specs=[pl.BlockSpec((1, 1, NG), lambda b: (b, 0, 0))],
        out_specs=pl.BlockSpec((1, 1, M), lambda b: (b, 0, 0)),
        out_shape=jax.ShapeDtypeStruct((B, 1, M), jnp.int32),
    )(idx)


def _gates_body(ka_ref, va_ref, q_ref, mom_ref, ln_ref,
                wk0_ref, wk1_ref, bk_ref, wv0_ref, wv1_ref, bv_ref,
                leta_ref, lalpha_ref, gk_ref, gv_ref, nm_ref):
    scale = D ** (-0.5)
    q = q_ref[0, 0, :]

    def surprise(a_ref):
        a = a_ref[0]
        logit = jnp.dot(a, q, preferred_element_type=jnp.float32) * scale
        e = jnp.exp(logit - jnp.max(logit))
        attn = e / jnp.sum(e)
        pred = jnp.dot(attn, a, preferred_element_type=jnp.float32)
        return jnp.mean((pred - q) ** 2)

    ks = surprise(ka_ref)
    vs = surprise(va_ref)
    alpha = jax.nn.sigmoid(lalpha_ref[0, 0, 0])
    comb = alpha * ks + (1.0 - alpha) * vs
    eta = jax.nn.sigmoid(leta_ref[0, 0, 0])
    nm = eta * mom_ref[0, 0, 0] + (1.0 - eta) * comb
    nm_ref[0] = jnp.full((1, 1), nm, jnp.float32)

    ln1 = ln_ref[0, 0, :]
    ion1 = lax.broadcasted_iota(jnp.int32, (M, NG), 1) + 1
    Bm = (ln1[:, None] == ion1).astype(jnp.float32)
    inactive = (ln1 == 0).astype(jnp.float32)

    def one(w0_ref, w1_ref, b_ref, g_ref):
        gate_n = jax.nn.sigmoid(ks * w0_ref[0, 0, :] + nm * w1_ref[0, 0, :]
                                + b_ref[0, 0, :])
        g_ref[0, 0, :] = jnp.dot(Bm, gate_n,
                                 preferred_element_type=jnp.float32) + inactive

    one(wk0_ref, wk1_ref, bk_ref, gk_ref)
    one(wv0_ref, wv1_ref, bv_ref, gv_ref)


def _gates(Ka, Va, q, mom, ln, wk0, wk1, bk, wv0, wv1, bv, leta, lalpha):
    bcast = pl.BlockSpec((1, 1, NG), lambda b: (0, 0, 0))
    scal = pl.BlockSpec((1, 1, 1), lambda b: (0, 0, 0))
    return pl.pallas_call(
        _gates_body,
        grid=(B,),
        in_specs=[
            pl.BlockSpec((1, NG, D), lambda b: (b, 0, 0)),
            pl.BlockSpec((1, NG, D), lambda b: (b, 0, 0)),
            pl.BlockSpec((1, 1, D), lambda b: (b, 0, 0)),
            pl.BlockSpec((1, 1, 1), lambda b: (b, 0, 0)),
            pl.BlockSpec((1, 1, M), lambda b: (b, 0, 0)),
            bcast, bcast, bcast, bcast, bcast, bcast,
            scal, scal,
        ],
        out_specs=[
            pl.BlockSpec((1, 1, M), lambda b: (b, 0, 0)),
            pl.BlockSpec((1, 1, M), lambda b: (b, 0, 0)),
            pl.BlockSpec((1, 1, 1), lambda b: (b, 0, 0)),
        ],
        out_shape=[
            jax.ShapeDtypeStruct((B, 1, M), jnp.float32),
            jax.ShapeDtypeStruct((B, 1, M), jnp.float32),
            jax.ShapeDtypeStruct((B, 1, 1), jnp.float32),
        ],
    )(Ka, Va, q, mom, ln, wk0, wk1, bk, wv0, wv1, bv, leta, lalpha)


BM_BLEND = 1024


def _blend_body(kc_ref, kp_ref, vc_ref, vp_ref, gk_ref, gv_ref,
                ko_ref, vo_ref):
    j = pl.program_id(1)
    gk = gk_ref[0, 0, pl.ds(j * BM_BLEND, BM_BLEND)][:, None]
    gv = gv_ref[0, 0, pl.ds(j * BM_BLEND, BM_BLEND)][:, None]
    ko_ref[0] = kc_ref[0] * gk + kp_ref[0] * (1.0 - gk)
    vo_ref[0] = vc_ref[0] * gv + vp_ref[0] * (1.0 - gv)


def _blend(K_curr, K_prev, V_curr, V_prev, gk, gv):
    big = pl.BlockSpec((1, BM_BLEND, D), lambda b, j: (b, j, 0))
    row = pl.BlockSpec((1, 1, M), lambda b, j: (b, 0, 0))
    return pl.pallas_call(
        _blend_body,
        grid=(B, M // BM_BLEND),
        in_specs=[big, big, big, big, row, row],
        out_specs=[big, big],
        out_shape=[
            jax.ShapeDtypeStruct((B, M, D), jnp.float32),
            jax.ShapeDtypeStruct((B, M, D), jnp.float32),
        ],
    )(K_curr, K_prev, V_curr, V_prev, gk, gv)


def kernel(K_curr, V_curr, K_prev, V_prev, h, momentum, active_idx,
           Wk, bk, Wv, bv, logit_eta, surprise_logit_alpha):
    idx32 = active_idx.astype(jnp.int32)
    idx = idx32.reshape(B, 1, NG)
    Kf = K_curr.reshape(B * M, D)
    Vf = V_curr.reshape(B * M, D)
    idx4 = (idx32 + (jnp.arange(B, dtype=jnp.int32) * M)[:, None]
            ).reshape(B, NG // CR, CR)
    Ko, Vo = _sc_gather(Kf, Vf, idx4)
    ln = _lastn(idx)
    q = _qprobe(h)
    wk0 = Wk[:, 0].reshape(1, 1, NG)
    wk1 = Wk[:, 1].reshape(1, 1, NG)
    wv0 = Wv[:, 0].reshape(1, 1, NG)
    wv1 = Wv[:, 1].reshape(1, 1, NG)
    gk, gv, nm = _gates(Ko.reshape(B, NG, D), Vo.reshape(B, NG, D), q,
                        momentum.reshape(B, 1, 1), ln,
                        wk0, wk1, bk.reshape(1, 1, NG),
                        wv0, wv1, bv.reshape(1, 1, NG),
                        jnp.reshape(logit_eta, (1, 1, 1)),
                        jnp.reshape(surprise_logit_alpha, (1, 1, 1)))
    K_out, V_out = _blend(K_curr, K_prev, V_curr, V_prev, gk, gv)
    return (K_out, V_out, nm.reshape(B, 1))
